# speedup vs baseline: 1.5871x; 1.5871x over previous
import functools

import jax
import jax.numpy as jnp
from jax import lax
from jax.experimental import pallas as pl
from jax.experimental.pallas import tpu as pltpu
from jax.experimental.pallas import tpu_sc as plsc

_info = plsc.get_sparse_core_info()
NC = _info.num_cores
NS = _info.num_subcores
NW = NC * NS

CHUNK = 64


def _make_gather(batch_total: int, d_model: int):
    assert batch_total % NW == 0
    b_per_w = batch_total // NW
    assert b_per_w % CHUNK == 0
    n_chunks = b_per_w // CHUNK

    mesh = plsc.VectorSubcoreMesh(core_axis_name="c", subcore_axis_name="s")

    @functools.partial(
        pl.kernel,
        mesh=mesh,
        out_type=jax.ShapeDtypeStruct((batch_total, d_model), jnp.float32),
        scratch_types=[
            pltpu.VMEM((n_chunks, CHUNK), jnp.int32),
            pltpu.VMEM((CHUNK, d_model), jnp.float32),
            pltpu.SemaphoreType.DMA,
        ],
    )
    def gather_k(ids_hbm, table_hbm, out_hbm, idx_v, rows_v, sem):
        wid = lax.axis_index("s") * NC + lax.axis_index("c")
        pltpu.sync_copy(ids_hbm.at[wid], idx_v)
        base = wid * b_per_w
        for j in range(n_chunks):
            pltpu.async_copy(table_hbm.at[idx_v.at[j]], rows_v, sem).wait()
            pltpu.sync_copy(rows_v, out_hbm.at[pl.ds(base + j * CHUNK, CHUNK)])

    return gather_k


def kernel(input_ids, word_embeddings):
    batch, seqlen = input_ids.shape
    total = batch * seqlen
    d_model = word_embeddings.shape[1]
    b_per_w = total // NW
    ids = input_ids.reshape(NW, b_per_w // CHUNK, CHUNK)
    out = _make_gather(total, d_model)(ids, word_embeddings)
    return out.reshape(batch, seqlen, d_model)


# speedup vs baseline: 1.6739x; 1.0547x over previous
import functools

import jax
import jax.numpy as jnp
from jax import lax
from jax.experimental import pallas as pl
from jax.experimental.pallas import tpu as pltpu
from jax.experimental.pallas import tpu_sc as plsc

_info = plsc.get_sparse_core_info()
NC = _info.num_cores
NS = _info.num_subcores
NW = NC * NS

CHUNK = 32


def _make_gather(batch_total: int, d_model: int):
    assert batch_total % NW == 0
    b_per_w = batch_total // NW
    assert b_per_w % (2 * CHUNK) == 0
    n_chunks = b_per_w // CHUNK
    n_pairs = n_chunks // 2

    mesh = plsc.VectorSubcoreMesh(core_axis_name="c", subcore_axis_name="s")

    @functools.partial(
        pl.kernel,
        mesh=mesh,
        out_type=jax.ShapeDtypeStruct((batch_total, d_model), jnp.float32),
        scratch_types=[
            pltpu.VMEM((n_chunks, CHUNK), jnp.int32),
            pltpu.VMEM((CHUNK, d_model), jnp.float32),
            pltpu.VMEM((CHUNK, d_model), jnp.float32),
            pltpu.SemaphoreType.DMA,
            pltpu.SemaphoreType.DMA,
            pltpu.SemaphoreType.DMA,
            pltpu.SemaphoreType.DMA,
        ],
    )
    def gather_k(ids_hbm, table_hbm, out_hbm, idx_v, buf0, buf1,
                 gsem0, gsem1, osem0, osem1):
        wid = lax.axis_index("s") * NC + lax.axis_index("c")
        pltpu.sync_copy(ids_hbm.at[wid], idx_v)
        base = wid * b_per_w

        def out_at(c):
            return out_hbm.at[pl.ds(base + c * CHUNK, CHUNK)]

        g0 = pltpu.async_copy(table_hbm.at[idx_v.at[0]], buf0, gsem0)
        g1 = pltpu.async_copy(table_hbm.at[idx_v.at[1]], buf1, gsem1)
        g0.wait()
        pltpu.async_copy(buf0, out_at(0), osem0)
        g1.wait()
        pltpu.async_copy(buf1, out_at(1), osem1)

        @pl.loop(1, n_pairs)
        def _pair(j):
            c0 = j * 2
            c1 = c0 + 1
            pltpu.make_async_copy(buf0, out_at(c0 - 2), osem0).wait()
            ga = pltpu.async_copy(table_hbm.at[idx_v.at[c0]], buf0, gsem0)
            pltpu.make_async_copy(buf1, out_at(c1 - 2), osem1).wait()
            gb = pltpu.async_copy(table_hbm.at[idx_v.at[c1]], buf1, gsem1)
            ga.wait()
            pltpu.async_copy(buf0, out_at(c0), osem0)
            gb.wait()
            pltpu.async_copy(buf1, out_at(c1), osem1)

        pltpu.make_async_copy(buf0, out_at(n_chunks - 2), osem0).wait()
        pltpu.make_async_copy(buf1, out_at(n_chunks - 1), osem1).wait()

    return gather_k


def kernel(input_ids, word_embeddings):
    batch, seqlen = input_ids.shape
    total = batch * seqlen
    d_model = word_embeddings.shape[1]
    b_per_w = total // NW
    ids = input_ids.reshape(NW, b_per_w // CHUNK, CHUNK)
    out = _make_gather(total, d_model)(ids, word_embeddings)
    return out.reshape(batch, seqlen, d_model)


# speedup vs baseline: 1.7270x; 1.0317x over previous
import functools

import jax
import jax.numpy as jnp
from jax import lax
from jax.experimental import pallas as pl
from jax.experimental.pallas import tpu as pltpu
from jax.experimental.pallas import tpu_sc as plsc

_info = plsc.get_sparse_core_info()
NC = _info.num_cores
NS = _info.num_subcores
NW = NC * NS

CHUNK = 16
NBUF = 4


def _make_gather(batch_total: int, d_model: int):
    assert batch_total % NW == 0
    b_per_w = batch_total // NW
    assert b_per_w % (NBUF * CHUNK) == 0
    n_chunks = b_per_w // CHUNK
    n_groups = n_chunks // NBUF

    mesh = plsc.VectorSubcoreMesh(core_axis_name="c", subcore_axis_name="s")

    @functools.partial(
        pl.kernel,
        mesh=mesh,
        out_type=jax.ShapeDtypeStruct((batch_total, d_model), jnp.float32),
        scratch_types=[
            pltpu.VMEM((n_chunks, CHUNK), jnp.int32),
        ] + [pltpu.VMEM((CHUNK, d_model), jnp.float32)] * NBUF
          + [pltpu.SemaphoreType.DMA] * (2 * NBUF),
    )
    def gather_k(ids_hbm, table_hbm, out_hbm, idx_v, *rest):
        bufs = rest[:NBUF]
        gsems = rest[NBUF:2 * NBUF]
        osems = rest[2 * NBUF:]
        wid = lax.axis_index("s") * NC + lax.axis_index("c")
        pltpu.sync_copy(ids_hbm.at[wid], idx_v)
        base = wid * b_per_w

        def out_at(c):
            return out_hbm.at[pl.ds(base + c * CHUNK, CHUNK)]

        for b in range(NBUF):
            pltpu.async_copy(table_hbm.at[idx_v.at[b]], bufs[b], gsems[b])
        for b in range(NBUF):
            pltpu.make_async_copy(table_hbm.at[idx_v.at[b]], bufs[b],
                                  gsems[b]).wait()
            pltpu.async_copy(bufs[b], out_at(b), osems[b])

        @pl.loop(1, n_groups)
        def _group(j):
            c = j * NBUF
            for b in range(NBUF):
                pltpu.make_async_copy(bufs[b], out_at(c + b - NBUF),
                                      osems[b]).wait()
                pltpu.async_copy(table_hbm.at[idx_v.at[c + b]], bufs[b],
                                 gsems[b])
            for b in range(NBUF):
                pltpu.make_async_copy(table_hbm.at[idx_v.at[c + b]], bufs[b],
                                      gsems[b]).wait()
                pltpu.async_copy(bufs[b], out_at(c + b), osems[b])

        for b in range(NBUF):
            pltpu.make_async_copy(bufs[b], out_at(n_chunks - NBUF + b),
                                  osems[b]).wait()

    return gather_k


def kernel(input_ids, word_embeddings):
    batch, seqlen = input_ids.shape
    total = batch * seqlen
    d_model = word_embeddings.shape[1]
    b_per_w = total // NW
    ids = input_ids.reshape(NW, b_per_w // CHUNK, CHUNK)
    out = _make_gather(total, d_model)(ids, word_embeddings)
    return out.reshape(batch, seqlen, d_model)


# speedup vs baseline: 1.7332x; 1.0036x over previous
import functools

import jax
import jax.numpy as jnp
from jax import lax
from jax.experimental import pallas as pl
from jax.experimental.pallas import tpu as pltpu
from jax.experimental.pallas import tpu_sc as plsc

_info = plsc.get_sparse_core_info()
NC = _info.num_cores
NS = _info.num_subcores
NW = NC * NS

CHUNK = 16
NBUF = 6


def _make_gather(batch_total: int, d_model: int):
    assert batch_total % NW == 0
    b_per_w = batch_total // NW
    assert b_per_w % CHUNK == 0
    n_chunks = b_per_w // CHUNK
    n_main = n_chunks // NBUF
    tail = n_chunks - n_main * NBUF
    assert n_main >= 2

    mesh = plsc.VectorSubcoreMesh(core_axis_name="c", subcore_axis_name="s")

    @functools.partial(
        pl.kernel,
        mesh=mesh,
        out_type=jax.ShapeDtypeStruct((batch_total, d_model), jnp.float32),
        scratch_types=[
            pltpu.VMEM((n_chunks, CHUNK), jnp.int32),
        ] + [pltpu.VMEM((CHUNK, d_model), jnp.float32)] * NBUF
          + [pltpu.SemaphoreType.DMA] * (2 * NBUF),
    )
    def gather_k(ids_hbm, table_hbm, out_hbm, idx_v, *rest):
        bufs = rest[:NBUF]
        gsems = rest[NBUF:2 * NBUF]
        osems = rest[2 * NBUF:]
        wid = lax.axis_index("s") * NC + lax.axis_index("c")
        pltpu.sync_copy(ids_hbm.at[wid], idx_v)
        base = wid * b_per_w

        def out_at(c):
            return out_hbm.at[pl.ds(base + c * CHUNK, CHUNK)]

        def start_gather(c, b):
            pltpu.async_copy(table_hbm.at[idx_v.at[c]], bufs[b], gsems[b])

        def wait_gather(c, b):
            pltpu.make_async_copy(table_hbm.at[idx_v.at[c]], bufs[b],
                                  gsems[b]).wait()

        def start_out(c, b):
            pltpu.async_copy(bufs[b], out_at(c), osems[b])

        def wait_out(c, b):
            pltpu.make_async_copy(bufs[b], out_at(c), osems[b]).wait()

        for b in range(NBUF):
            start_gather(b, b)
        for b in range(NBUF):
            wait_gather(b, b)
            start_out(b, b)

        @pl.loop(1, n_main)
        def _group(j):
            c = j * NBUF
            for b in range(NBUF):
                wait_out(c + b - NBUF, b)
                start_gather(c + b, b)
            for b in range(NBUF):
                wait_gather(c + b, b)
                start_out(c + b, b)

        c0 = n_main * NBUF
        for b in range(tail):
            wait_out(c0 + b - NBUF, b)
            start_gather(c0 + b, b)
        for b in range(tail):
            wait_gather(c0 + b, b)
            start_out(c0 + b, b)

        for c in range(n_chunks - NBUF, n_chunks):
            wait_out(c, c % NBUF)

    return gather_k


def kernel(input_ids, word_embeddings):
    batch, seqlen = input_ids.shape
    total = batch * seqlen
    d_model = word_embeddings.shape[1]
    b_per_w = total // NW
    ids = input_ids.reshape(NW, b_per_w // CHUNK, CHUNK)
    out = _make_gather(total, d_model)(ids, word_embeddings)
    return out.reshape(batch, seqlen, d_model)
